# baseline (speedup 1.0000x reference)
import functools

import jax
import jax.numpy as jnp
from jax.experimental import pallas as pl
from jax.experimental.pallas import tpu as pltpu

_BN = 400
_NBUF = 10


def _gcn_body(x_hbm, s_ref, w_ref, b_ref, o_hbm,
              xbuf, obuf, c_ref, insems, outsems):
    nblk = x_hbm.shape[0] // _BN

    def in_copy(k):
        return pltpu.make_async_copy(
            x_hbm.at[pl.ds(k * _BN, _BN), :],
            xbuf.at[k % _NBUF],
            insems.at[k % _NBUF])

    def out_copy(k):
        return pltpu.make_async_copy(
            obuf.at[k % _NBUF],
            o_hbm.at[pl.ds(k * _BN, _BN), :],
            outsems.at[k % _NBUF])

    for k in range(_NBUF):
        in_copy(k).start()

    c32 = jax.lax.dot_general(
        s_ref[:], w_ref[:], (((1,), (1,)), ((), ())),
        preferred_element_type=jnp.float32)
    c_ref[:] = c32.astype(jnp.bfloat16)

    for k in range(nblk):
        slot = k % _NBUF
        in_copy(k).wait()
        acc = jnp.dot(xbuf[slot].astype(jnp.bfloat16), c_ref[:],
                      preferred_element_type=jnp.float32)
        res = jnp.maximum(acc + b_ref[:], 0.0)
        if k >= _NBUF:
            out_copy(k - _NBUF).wait()
        obuf[slot] = res
        out_copy(k).start()
        if k + _NBUF < nblk:
            in_copy(k + _NBUF).start()

    for k in range(nblk - _NBUF, nblk):
        out_copy(k).wait()


@functools.partial(jax.jit, static_argnames=())
def kernel(x, support, W, b):
    n, d = x.shape
    out_c, in_c = W.shape
    out = pl.pallas_call(
        _gcn_body,
        in_specs=[
            pl.BlockSpec(memory_space=pltpu.MemorySpace.HBM),
            pl.BlockSpec(memory_space=pltpu.MemorySpace.VMEM),
            pl.BlockSpec(memory_space=pltpu.MemorySpace.VMEM),
            pl.BlockSpec(memory_space=pltpu.MemorySpace.VMEM),
        ],
        out_specs=pl.BlockSpec(memory_space=pltpu.MemorySpace.HBM),
        out_shape=jax.ShapeDtypeStruct((n, out_c), jnp.float32),
        scratch_shapes=[
            pltpu.VMEM((_NBUF, _BN, d), jnp.float32),
            pltpu.VMEM((_NBUF, _BN, out_c), jnp.float32),
            pltpu.VMEM((d, out_c), jnp.bfloat16),
            pltpu.SemaphoreType.DMA((_NBUF,)),
            pltpu.SemaphoreType.DMA((_NBUF,)),
        ],
        compiler_params=pltpu.CompilerParams(
            vmem_limit_bytes=120 * 1024 * 1024),
    )(x, support, W, b.reshape(1, out_c))
    return out


# speedup vs baseline: 1.1454x; 1.1454x over previous
import functools

import jax
import jax.numpy as jnp
from jax.experimental import pallas as pl
from jax.experimental.pallas import tpu as pltpu

_BN = 2000
_NBUF = 5


def _gcn_body(x_hbm, s_ref, w_ref, b_ref, o_hbm,
              xbuf, obuf, c_ref, insems, outsems):
    nblk = x_hbm.shape[0] // _BN

    def in_copy(k):
        return pltpu.make_async_copy(
            x_hbm.at[pl.ds(k * _BN, _BN), :],
            xbuf.at[k % _NBUF],
            insems.at[k % _NBUF])

    def out_copy(k):
        return pltpu.make_async_copy(
            obuf.at[k % _NBUF],
            o_hbm.at[pl.ds(k * _BN, _BN), :],
            outsems.at[k % _NBUF])

    for k in range(_NBUF):
        in_copy(k).start()

    c32 = jax.lax.dot_general(
        s_ref[:], w_ref[:], (((1,), (1,)), ((), ())),
        preferred_element_type=jnp.float32)
    c_ref[:] = c32.astype(jnp.bfloat16)

    for k in range(nblk):
        slot = k % _NBUF
        in_copy(k).wait()
        acc = jnp.dot(xbuf[slot].astype(jnp.bfloat16), c_ref[:],
                      preferred_element_type=jnp.float32)
        res = jnp.maximum(acc + b_ref[:], 0.0)
        if k >= _NBUF:
            out_copy(k - _NBUF).wait()
        obuf[slot] = res
        out_copy(k).start()
        if k + _NBUF < nblk:
            in_copy(k + _NBUF).start()

    for k in range(nblk - _NBUF, nblk):
        out_copy(k).wait()


@functools.partial(jax.jit, static_argnames=())
def kernel(x, support, W, b):
    n, d = x.shape
    out_c, in_c = W.shape
    out = pl.pallas_call(
        _gcn_body,
        in_specs=[
            pl.BlockSpec(memory_space=pltpu.MemorySpace.HBM),
            pl.BlockSpec(memory_space=pltpu.MemorySpace.VMEM),
            pl.BlockSpec(memory_space=pltpu.MemorySpace.VMEM),
            pl.BlockSpec(memory_space=pltpu.MemorySpace.VMEM),
        ],
        out_specs=pl.BlockSpec(memory_space=pltpu.MemorySpace.HBM),
        out_shape=jax.ShapeDtypeStruct((n, out_c), jnp.float32),
        scratch_shapes=[
            pltpu.VMEM((_NBUF, _BN, d), jnp.float32),
            pltpu.VMEM((_NBUF, _BN, out_c), jnp.float32),
            pltpu.VMEM((d, out_c), jnp.bfloat16),
            pltpu.SemaphoreType.DMA((_NBUF,)),
            pltpu.SemaphoreType.DMA((_NBUF,)),
        ],
        compiler_params=pltpu.CompilerParams(
            vmem_limit_bytes=120 * 1024 * 1024),
    )(x, support, W, b.reshape(1, out_c))
    return out
